# baseline (speedup 1.0000x reference)
import ml_dtypes
import numpy as np
import jax
import jax.numpy as jnp
from jax import lax
from jax.experimental import pallas as pl
from jax.experimental.pallas import tpu as pltpu
from jax.experimental.pallas import tpu_sc as plsc

D_MODEL = 1024
LENGTH = 2048
BATCH = 4
SEQ = 2048

NC, NS = 2, 16
NW = NC * NS
POS_PER_W = SEQ // NW
CHUNK = 16
N_PC = POS_PER_W // CHUNK
N_IT = N_PC * BATCH
RING = 5
AHEAD = 3
LANES = 16
SCALE = 32.0


def _positional_encoding_np(length, depth):
    depth = depth / 2
    positions = np.arange(length)[:, np.newaxis]
    depths = np.arange(depth)[np.newaxis, :] / depth
    with np.errstate(divide="ignore", invalid="ignore"):
        angle_rates = 1 / (10000 * depths)
        angle_rads = positions * angle_rates
    return np.concatenate(
        [np.sin(angle_rads), np.cos(angle_rads)], axis=-1
    ).astype(np.float32)


_PE = _positional_encoding_np(LENGTH, D_MODEL)
_PE_I = (_PE.reshape(LENGTH, D_MODEL // 32, 2, 16)
         .transpose(0, 1, 3, 2).reshape(LENGTH, D_MODEL))
_PE_PACKED = np.ascontiguousarray(
    _PE_I.astype(ml_dtypes.bfloat16)).view(np.int32).reshape(
        LENGTH, D_MODEL // 2)


def _sc_body(table_hbm, idx_hbm, pe_hbm, out_hbm,
             idx_v, r0, r1, r2, r3, r4, pe0, pe1, sg, ss, spe):
    wid = lax.axis_index("s") * NC + lax.axis_index("c")
    p0 = wid * POS_PER_W
    rows = [r0, r1, r2, r3, r4]
    pe_bufs = [pe0, pe1]

    for b in range(BATCH):
        pltpu.sync_copy(idx_hbm.at[b, pl.ds(p0, POS_PER_W)],
                        idx_v.at[pl.ds(b * POS_PER_W, POS_PER_W)])

    def offs(it):
        pc, b = it // BATCH, it % BATCH
        return b, p0 + pc * CHUNK, b * POS_PER_W + pc * CHUNK

    def start_gather(it):
        _, _, idx_off = offs(it)
        k = it % RING
        return pltpu.async_copy(
            table_hbm.at[idx_v.at[pl.ds(idx_off, CHUNK)]], rows[k], sg.at[k])

    def start_pe(pc):
        j = pc % 2
        return pltpu.async_copy(
            pe_hbm.at[pl.ds(p0 + pc * CHUNK, CHUNK)], pe_bufs[j], spe.at[j])

    gathers = [None] * N_IT
    stores = [None] * N_IT
    pe_descs = [None] * N_PC
    gathers[0] = start_gather(0)
    pe_descs[0] = start_pe(0)
    for it in range(1, AHEAD):
        gathers[it] = start_gather(it)

    for it in range(N_IT):
        k = it % RING
        nxt = it + AHEAD
        if nxt < N_IT:
            if nxt - RING >= 0:
                stores[nxt - RING].wait()
            gathers[nxt] = start_gather(nxt)
        if it % BATCH == 0:
            pc = it // BATCH
            pe_descs[pc].wait()
            if pc + 1 < N_PC:
                pe_descs[pc + 1] = start_pe(pc + 1)
        pe_v = pe_bufs[(it // BATCH) % 2]
        gathers[it].wait()

        def row_step(r, carry, _k=k, _pe=pe_v):
            for j2 in range(D_MODEL // (2 * LANES)):
                pi = _pe[r, pl.ds(j2 * LANES, LANES)]
                pa = lax.bitcast_convert_type(pi << 16, jnp.float32)
                pb = lax.bitcast_convert_type(
                    pi & jnp.int32(-65536), jnp.float32)
                sa = pl.ds(j2 * 2 * LANES, LANES)
                sb = pl.ds(j2 * 2 * LANES + LANES, LANES)
                rows[_k][r, sa] = rows[_k][r, sa] * SCALE + pa
                rows[_k][r, sb] = rows[_k][r, sb] * SCALE + pb
            return carry

        lax.fori_loop(0, CHUNK, row_step, 0)
        b, pos, _ = offs(it)
        stores[it] = pltpu.async_copy(
            rows[k], out_hbm.at[b, pl.ds(pos, CHUNK)], ss.at[k])

    for it in range(N_IT - RING, N_IT):
        stores[it].wait()


@jax.jit
def _pos_embed(x2d, table, pe2d):
    mesh = plsc.VectorSubcoreMesh(core_axis_name="c", subcore_axis_name="s")
    fn = pl.kernel(
        _sc_body,
        out_type=jax.ShapeDtypeStruct((BATCH, SEQ, D_MODEL), jnp.float32),
        mesh=mesh,
        scratch_types=[
            pltpu.VMEM((BATCH * POS_PER_W,), jnp.int32),
        ] + [pltpu.VMEM((CHUNK, D_MODEL), jnp.float32)] * RING + [
            pltpu.VMEM((CHUNK, D_MODEL // 2), jnp.int32),
            pltpu.VMEM((CHUNK, D_MODEL // 2), jnp.int32),
            pltpu.SemaphoreType.DMA((RING,)),
            pltpu.SemaphoreType.DMA((RING,)),
            pltpu.SemaphoreType.DMA((2,)),
        ],
    )
    return fn(table, x2d, pe2d)


def kernel(x, table):
    x2d = x.astype(jnp.int32)
    pe_i32 = jnp.asarray(_PE_PACKED)
    return _pos_embed(x2d, table, pe_i32)


# speedup vs baseline: 1.0154x; 1.0154x over previous
import numpy as np
import jax
import jax.numpy as jnp
from jax import lax
from jax.experimental import pallas as pl
from jax.experimental.pallas import tpu as pltpu
from jax.experimental.pallas import tpu_sc as plsc

D_MODEL = 1024
LENGTH = 2048
BATCH = 4
SEQ = 2048

NC, NS = 2, 16
NW = NC * NS
POS_PER_W = SEQ // NW
CHUNK = 16
N_PC = POS_PER_W // CHUNK
N_IT = N_PC * BATCH
RING = 5
AHEAD = 3
LANES = 16
SCALE = 32.0


def _positional_encoding_np(length, depth):
    depth = depth / 2
    positions = np.arange(length)[:, np.newaxis]
    depths = np.arange(depth)[np.newaxis, :] / depth
    with np.errstate(divide="ignore", invalid="ignore"):
        angle_rates = 1 / (10000 * depths)
        angle_rads = positions * angle_rates
    return np.concatenate(
        [np.sin(angle_rads), np.cos(angle_rads)], axis=-1
    ).astype(np.float32)


_PE = _positional_encoding_np(LENGTH, D_MODEL)


def _sc_body(table_hbm, idx_hbm, pe_hbm, out_hbm,
             idx_v, rows_all, pe_all, sems):
    wid = lax.axis_index("s") * NC + lax.axis_index("c")
    p0 = wid * POS_PER_W
    rows = [rows_all.at[k] for k in range(RING)]
    pe_bufs = [pe_all.at[j] for j in range(2)]
    sg = [sems.at[k] for k in range(RING)]
    ss = [sems.at[RING + k] for k in range(RING)]
    spe = [sems.at[2 * RING + j] for j in range(2)]

    for b in range(BATCH):
        pltpu.sync_copy(idx_hbm.at[b, pl.ds(p0, POS_PER_W)],
                        idx_v.at[pl.ds(b * POS_PER_W, POS_PER_W)])

    def offs(it):
        pc, b = it // BATCH, it % BATCH
        return b, p0 + pc * CHUNK, b * POS_PER_W + pc * CHUNK

    def start_gather(it):
        _, _, idx_off = offs(it)
        k = it % RING
        return pltpu.async_copy(
            table_hbm.at[idx_v.at[pl.ds(idx_off, CHUNK)]], rows[k], sg[k])

    def start_pe(pc):
        j = pc % 2
        return pltpu.async_copy(
            pe_hbm.at[pl.ds(p0 + pc * CHUNK, CHUNK)], pe_bufs[j], spe[j])

    gathers = [None] * N_IT
    stores = [None] * N_IT
    pe_descs = [None] * N_PC
    gathers[0] = start_gather(0)
    pe_descs[0] = start_pe(0)
    for it in range(1, AHEAD):
        gathers[it] = start_gather(it)

    for it in range(N_IT):
        k = it % RING
        nxt = it + AHEAD
        if nxt < N_IT:
            if nxt - RING >= 0:
                stores[nxt - RING].wait()
            gathers[nxt] = start_gather(nxt)
        if it % BATCH == 0:
            pc = it // BATCH
            pe_descs[pc].wait()
            if pc + 1 < N_PC:
                pe_descs[pc + 1] = start_pe(pc + 1)
        pe_v = pe_bufs[(it // BATCH) % 2]
        gathers[it].wait()

        def row_step(r, carry, _k=k, _j=(it // BATCH) % 2):
            for j in range(D_MODEL // LANES):
                s = pl.ds(j * LANES, LANES)
                rows_all[_k, r, s] = (rows_all[_k, r, s] * SCALE
                                      + pe_all[_j, r, s])
            return carry

        lax.fori_loop(0, CHUNK, row_step, 0)
        b, pos, _ = offs(it)
        stores[it] = pltpu.async_copy(
            rows[k], out_hbm.at[b, pl.ds(pos, CHUNK)], ss[k])

    for it in range(N_IT - RING, N_IT):
        stores[it].wait()


@jax.jit
def _pos_embed(x2d, table, pe2d):
    mesh = plsc.VectorSubcoreMesh(core_axis_name="c", subcore_axis_name="s")
    fn = pl.kernel(
        _sc_body,
        out_type=jax.ShapeDtypeStruct((BATCH, SEQ, D_MODEL), jnp.float32),
        mesh=mesh,
        scratch_types=[
            pltpu.VMEM((BATCH * POS_PER_W,), jnp.int32),
            pltpu.VMEM((RING, CHUNK, D_MODEL), jnp.float32),
            pltpu.VMEM((2, CHUNK, D_MODEL), jnp.float32),
            pltpu.SemaphoreType.DMA((2 * RING + 2,)),
        ],
    )
    return fn(table, x2d, pe2d)


def kernel(x, table):
    x2d = x.astype(jnp.int32)
    pe2d = jnp.asarray(_PE)
    return _pos_embed(x2d, table, pe2d)


# speedup vs baseline: 1.3673x; 1.3465x over previous
import numpy as np
import jax
import jax.numpy as jnp
from jax import lax
from jax.experimental import pallas as pl
from jax.experimental.pallas import tpu as pltpu
from jax.experimental.pallas import tpu_sc as plsc

D_MODEL = 1024
LENGTH = 2048
BATCH = 4
SEQ = 2048

NC, NS = 2, 16
NW = NC * NS
POS_PER_W = SEQ // NW
CHUNK = 16
N_PC = POS_PER_W // CHUNK
N_IT = N_PC * BATCH
RING = 5
AHEAD = 2
LANES = 16
SCALE = 32.0


def _positional_encoding_np(length, depth):
    depth = depth / 2
    positions = np.arange(length)[:, np.newaxis]
    depths = np.arange(depth)[np.newaxis, :] / depth
    with np.errstate(divide="ignore", invalid="ignore"):
        angle_rates = 1 / (10000 * depths)
        angle_rads = positions * angle_rates
    return np.concatenate(
        [np.sin(angle_rads), np.cos(angle_rads)], axis=-1
    ).astype(np.float32)


_PE = _positional_encoding_np(LENGTH, D_MODEL)


def _sc_body(table_hbm, idx_hbm, pe_hbm, out_hbm,
             idx_v, r0, r1, r2, r3, r4, pe0, pe1, sg, ss, spe):
    wid = lax.axis_index("s") * NC + lax.axis_index("c")
    p0 = wid * POS_PER_W
    rows = [r0, r1, r2, r3, r4]
    pe_bufs = [pe0, pe1]

    for b in range(BATCH):
        pltpu.sync_copy(idx_hbm.at[b, pl.ds(p0, POS_PER_W)],
                        idx_v.at[pl.ds(b * POS_PER_W, POS_PER_W)])

    def offs(it):
        pc, b = it // BATCH, it % BATCH
        return b, p0 + pc * CHUNK, b * POS_PER_W + pc * CHUNK

    def start_gather(it):
        _, _, idx_off = offs(it)
        k = it % RING
        return pltpu.async_copy(
            table_hbm.at[idx_v.at[pl.ds(idx_off, CHUNK)]], rows[k], sg.at[k])

    def start_pe(pc):
        j = pc % 2
        return pltpu.async_copy(
            pe_hbm.at[pl.ds(p0 + pc * CHUNK, CHUNK)], pe_bufs[j], spe.at[j])

    gathers = [None] * N_IT
    stores = [None] * N_IT
    pe_descs = [None] * N_PC
    gathers[0] = start_gather(0)
    pe_descs[0] = start_pe(0)
    for it in range(1, AHEAD):
        gathers[it] = start_gather(it)

    for it in range(N_IT):
        k = it % RING
        nxt = it + AHEAD
        if nxt < N_IT:
            if nxt - RING >= 0:
                stores[nxt - RING].wait()
            gathers[nxt] = start_gather(nxt)
        if it % BATCH == 0:
            pc = it // BATCH
            pe_descs[pc].wait()
            if pc + 1 < N_PC:
                pe_descs[pc + 1] = start_pe(pc + 1)
        pe_v = pe_bufs[(it // BATCH) % 2]
        gathers[it].wait()

        def row_step(r, carry, _k=k, _pe=pe_v):
            for j in range(D_MODEL // LANES):
                s = pl.ds(j * LANES, LANES)
                rows[_k][r, s] = rows[_k][r, s] * SCALE + _pe[r, s]
            return carry

        lax.fori_loop(0, CHUNK, row_step, 0)
        b, pos, _ = offs(it)
        stores[it] = pltpu.async_copy(
            rows[k], out_hbm.at[b, pl.ds(pos, CHUNK)], ss.at[k])

    for it in range(N_IT - RING, N_IT):
        stores[it].wait()


@jax.jit
def _pos_embed(x2d, table, pe2d):
    mesh = plsc.VectorSubcoreMesh(core_axis_name="c", subcore_axis_name="s")
    fn = pl.kernel(
        _sc_body,
        out_type=jax.ShapeDtypeStruct((BATCH, SEQ, D_MODEL), jnp.float32),
        mesh=mesh,
        scratch_types=[
            pltpu.VMEM((BATCH * POS_PER_W,), jnp.int32),
        ] + [pltpu.VMEM((CHUNK, D_MODEL), jnp.float32)] * RING + [
            pltpu.VMEM((CHUNK, D_MODEL), jnp.float32),
            pltpu.VMEM((CHUNK, D_MODEL), jnp.float32),
            pltpu.SemaphoreType.DMA((RING,)),
            pltpu.SemaphoreType.DMA((RING,)),
            pltpu.SemaphoreType.DMA((2,)),
        ],
    )
    return fn(table, x2d, pe2d)


def kernel(x, table):
    x2d = x.astype(jnp.int32)
    pe2d = jnp.asarray(_PE)
    return _pos_embed(x2d, table, pe2d)


# speedup vs baseline: 1.3892x; 1.0160x over previous
import ml_dtypes
import numpy as np
import jax
import jax.numpy as jnp
from jax import lax
from jax.experimental import pallas as pl
from jax.experimental.pallas import tpu as pltpu
from jax.experimental.pallas import tpu_sc as plsc

D_MODEL = 1024
LENGTH = 2048
BATCH = 4
SEQ = 2048

NC, NS = 2, 16
NW = NC * NS
POS_PER_W = SEQ // NW
CHUNK = 16
N_PC = POS_PER_W // CHUNK
N_IT = N_PC * BATCH
RING = 5
AHEAD = 2
LANES = 16
SCALE = 32.0


def _positional_encoding_np(length, depth):
    depth = depth / 2
    positions = np.arange(length)[:, np.newaxis]
    depths = np.arange(depth)[np.newaxis, :] / depth
    with np.errstate(divide="ignore", invalid="ignore"):
        angle_rates = 1 / (10000 * depths)
        angle_rads = positions * angle_rates
    return np.concatenate(
        [np.sin(angle_rads), np.cos(angle_rads)], axis=-1
    ).astype(np.float32)


_PE = _positional_encoding_np(LENGTH, D_MODEL)
_PE_BF = _PE.astype(ml_dtypes.bfloat16)


def _sc_body(table_hbm, idx_hbm, pe_hbm, out_hbm,
             idx_v, r0, r1, r2, r3, r4, pe0, pe1, sg, ss, spe):
    wid = lax.axis_index("s") * NC + lax.axis_index("c")
    p0 = wid * POS_PER_W
    rows = [r0, r1, r2, r3, r4]
    pe_bufs = [pe0, pe1]

    for b in range(BATCH):
        pltpu.sync_copy(idx_hbm.at[b, pl.ds(p0, POS_PER_W)],
                        idx_v.at[pl.ds(b * POS_PER_W, POS_PER_W)])

    def offs(it):
        pc, b = it // BATCH, it % BATCH
        return b, p0 + pc * CHUNK, b * POS_PER_W + pc * CHUNK

    def start_gather(it):
        _, _, idx_off = offs(it)
        k = it % RING
        return pltpu.async_copy(
            table_hbm.at[idx_v.at[pl.ds(idx_off, CHUNK)]], rows[k], sg.at[k])

    def start_pe(pc):
        j = pc % 2
        return pltpu.async_copy(
            pe_hbm.at[pl.ds(p0 + pc * CHUNK, CHUNK)], pe_bufs[j], spe.at[j])

    gathers = [None] * N_IT
    stores = [None] * N_IT
    pe_descs = [None] * N_PC
    gathers[0] = start_gather(0)
    pe_descs[0] = start_pe(0)
    for it in range(1, AHEAD):
        gathers[it] = start_gather(it)

    for it in range(N_IT):
        k = it % RING
        nxt = it + AHEAD
        if nxt < N_IT:
            if nxt - RING >= 0:
                stores[nxt - RING].wait()
            gathers[nxt] = start_gather(nxt)
        if it % BATCH == 0:
            pc = it // BATCH
            pe_descs[pc].wait()
            if pc + 1 < N_PC:
                pe_descs[pc + 1] = start_pe(pc + 1)
        pe_v = pe_bufs[(it // BATCH) % 2]
        gathers[it].wait()

        def row_step(r, carry, _k=k, _pe=pe_v):
            for j in range(D_MODEL // LANES):
                s = pl.ds(j * LANES, LANES)
                rows[_k][r, s] = rows[_k][r, s] * SCALE + _pe[r, s]
            return carry

        lax.fori_loop(0, CHUNK, row_step, 0)
        b, pos, _ = offs(it)
        stores[it] = pltpu.async_copy(
            rows[k], out_hbm.at[b, pl.ds(pos, CHUNK)], ss.at[k])

    for it in range(N_IT - RING, N_IT):
        stores[it].wait()


@jax.jit
def _pos_embed(x2d, table, pe2d):
    mesh = plsc.VectorSubcoreMesh(core_axis_name="c", subcore_axis_name="s")
    fn = pl.kernel(
        _sc_body,
        out_type=jax.ShapeDtypeStruct((BATCH, SEQ, D_MODEL), jnp.float32),
        mesh=mesh,
        scratch_types=[
            pltpu.VMEM((BATCH * POS_PER_W,), jnp.int32),
        ] + [pltpu.VMEM((CHUNK, D_MODEL), jnp.float32)] * RING + [
            pltpu.VMEM((CHUNK, D_MODEL), jnp.float32),
            pltpu.VMEM((CHUNK, D_MODEL), jnp.float32),
            pltpu.SemaphoreType.DMA((RING,)),
            pltpu.SemaphoreType.DMA((RING,)),
            pltpu.SemaphoreType.DMA((2,)),
        ],
    )
    return fn(table, x2d, pe2d)


def kernel(x, table):
    x2d = x.astype(jnp.int32)
    pe_bf = jax.lax.optimization_barrier(jnp.asarray(_PE_BF))
    pe2d = pe_bf.astype(jnp.float32)
    return _pos_embed(x2d, table, pe2d)
